# baseline (speedup 1.0000x reference)
import functools
import numpy as np
import jax
import jax.numpy as jnp
from jax.experimental import pallas as pl
from jax.experimental.pallas import tpu as pltpu
from jax.experimental.pallas import tpu_sc as plsc

_THRESH = float(-np.log(0.7))
_NMIN_FRAC = 0.1
_IGNORE = 255

_BH = 64


def _ce_loss_tile(z_ref, lab_ref):
    C = z_ref.shape[1]
    lab = lab_ref[0]
    m = z_ref[0, 0]
    for c in range(1, C):
        m = jnp.maximum(m, z_ref[0, c])
    s = jnp.zeros_like(m)
    picked = jnp.zeros_like(m)
    for c in range(C):
        zc = z_ref[0, c]
        s = s + jnp.exp(zc - m)
        picked = jnp.where(lab == c, zc, picked)
    loss = m + jnp.log(s) - picked
    return jnp.where(lab == _IGNORE, 0.0, loss)


def _ce_stats_body(z_ref, lab_ref, out_ref):
    loss = _ce_loss_tile(z_ref, lab_ref)
    mask = loss > _THRESH
    c = jnp.sum(mask.astype(jnp.float32))
    sm = jnp.sum(jnp.where(mask, loss, 0.0))
    first = (pl.program_id(0) == 0) & (pl.program_id(1) == 0)

    @pl.when(first)
    def _():
        out_ref[0] = 0.0
        out_ref[1] = 0.0

    out_ref[0] += c
    out_ref[1] += sm


def _ce_loss_body(z_ref, lab_ref, out_ref):
    out_ref[0] = _ce_loss_tile(z_ref, lab_ref)


_SC_NC = 2
_SC_NS = 16
_SC_NW = _SC_NC * _SC_NS
_SC_L = 16


@functools.cache
def _make_sc_countsum(n):
    per_w = n // _SC_NW
    steps = per_w // _SC_L
    mesh = plsc.VectorSubcoreMesh(core_axis_name="c", subcore_axis_name="s")

    @functools.partial(
        pl.kernel,
        mesh=mesh,
        out_type=jax.ShapeDtypeStruct((2, _SC_NW, _SC_L), jnp.float32),
        scratch_types=[
            pltpu.VMEM((per_w,), jnp.float32),
            pltpu.VMEM((_SC_L,), jnp.float32),
        ],
    )
    def countsum(loss_hbm, t_hbm, out_hbm, chunk, vec):
        cid = jax.lax.axis_index("c")
        sid = jax.lax.axis_index("s")
        wid = sid * _SC_NC + cid
        pltpu.sync_copy(loss_hbm.at[pl.ds(wid * per_w, per_w)], chunk)
        pltpu.sync_copy(t_hbm, vec)
        t = vec[...]
        thr = jnp.full((_SC_L,), _THRESH, jnp.float32)
        zero = jnp.zeros((_SC_L,), jnp.float32)
        one = jnp.full((_SC_L,), 1.0, jnp.float32)

        def body(i, carry):
            c_acc, s_acc = carry
            x = chunk[pl.ds(i * _SC_L, _SC_L)]
            keep = (x <= thr) & (x > t)
            return (
                c_acc + jnp.where(keep, one, zero),
                s_acc + jnp.where(keep, x, zero),
            )

        c_acc, s_acc = jax.lax.fori_loop(0, steps, body, (zero, zero))
        vec[...] = c_acc
        pltpu.sync_copy(vec, out_hbm.at[0, wid])
        vec[...] = s_acc
        pltpu.sync_copy(vec, out_hbm.at[1, wid])

    return countsum


def _run_ce_stats(logits, labels):
    B, C, H, W = logits.shape
    return pl.pallas_call(
        _ce_stats_body,
        grid=(B, H // _BH),
        in_specs=[
            pl.BlockSpec((1, C, _BH, W), lambda b, h: (b, 0, h, 0)),
            pl.BlockSpec((1, _BH, W), lambda b, h: (b, h, 0)),
        ],
        out_specs=pl.BlockSpec(memory_space=pltpu.SMEM),
        out_shape=jax.ShapeDtypeStruct((2,), jnp.float32),
        compiler_params=pltpu.CompilerParams(
            dimension_semantics=("arbitrary", "arbitrary")
        ),
    )(logits, labels)


def _topk_mean(logits, labels, cnt, ssum, n_min):
    B, C, H, W = logits.shape
    loss = pl.pallas_call(
        _ce_loss_body,
        grid=(B, H // _BH),
        in_specs=[
            pl.BlockSpec((1, C, _BH, W), lambda b, h: (b, 0, h, 0)),
            pl.BlockSpec((1, _BH, W), lambda b, h: (b, h, 0)),
        ],
        out_specs=pl.BlockSpec((1, _BH, W), lambda b, h: (b, h, 0)),
        out_shape=jax.ShapeDtypeStruct((B, H, W), jnp.float32),
        compiler_params=pltpu.CompilerParams(
            dimension_semantics=("arbitrary", "arbitrary")
        ),
    )(logits, labels)
    loss_flat = loss.reshape(B * H * W)
    kp = jnp.float32(n_min) - cnt
    countsum = _make_sc_countsum(B * H * W)

    def it(_, carry):
        lo, hi = carry
        mid = 0.5 * (lo + hi)
        part = countsum(loss_flat, jnp.broadcast_to(mid, (_SC_L,)))
        f = jnp.sum(part[0])
        gt = f > kp
        return jnp.where(gt, mid, lo), jnp.where(gt, hi, mid)

    _, hi = jax.lax.fori_loop(
        0, 50, it, (jnp.float32(-1.0), jnp.float32(_THRESH))
    )
    part = countsum(loss_flat, jnp.broadcast_to(hi, (_SC_L,)))
    fhi = jnp.sum(part[0])
    shi = jnp.sum(part[1])
    rest = shi + (kp - fhi) * hi
    return (ssum + rest) / jnp.float32(n_min)


def kernel(logits, labels):
    B, C, H, W = logits.shape
    labels = labels.astype(jnp.int32)
    n = B * H * W
    n_min = int(_NMIN_FRAC * n)
    stats = _run_ce_stats(logits, labels)
    cnt, ssum = stats[0], stats[1]
    mean_thresh = ssum / jnp.maximum(cnt, 1.0)
    return jax.lax.cond(
        cnt > jnp.float32(n_min),
        lambda: mean_thresh,
        lambda: _topk_mean(logits, labels, cnt, ssum, n_min),
    )


# speedup vs baseline: 1.1543x; 1.1543x over previous
import functools
import numpy as np
import jax
import jax.numpy as jnp
from jax.experimental import pallas as pl
from jax.experimental.pallas import tpu as pltpu
from jax.experimental.pallas import tpu_sc as plsc

_THRESH = float(-np.log(0.7))
_NMIN_FRAC = 0.1
_IGNORE = 255

_BH = 64


def _ce_loss_tile(z_ref, lab_ref):
    C = z_ref.shape[1]
    lab = lab_ref[0]
    m = z_ref[0, 0]
    for c in range(1, C):
        m = jnp.maximum(m, z_ref[0, c])
    s = jnp.zeros_like(m)
    picked = jnp.zeros_like(m)
    for c in range(C):
        zc = z_ref[0, c]
        s = s + jnp.exp(zc - m)
        picked = jnp.where(lab == c, zc, picked)
    loss = m + jnp.log(s) - picked
    return jnp.where(lab == _IGNORE, 0.0, loss)


def _ce_stats_body(z_ref, lab_ref, out_ref):
    loss = _ce_loss_tile(z_ref, lab_ref)
    mask = loss > _THRESH
    c = jnp.sum(mask.astype(jnp.float32))
    sm = jnp.sum(jnp.where(mask, loss, 0.0))
    first = (pl.program_id(0) == 0) & (pl.program_id(1) == 0)

    @pl.when(first)
    def _():
        out_ref[0] = 0.0
        out_ref[1] = 0.0

    out_ref[0] += c
    out_ref[1] += sm


def _ce_loss_body(z_ref, lab_ref, out_ref):
    out_ref[0] = _ce_loss_tile(z_ref, lab_ref)


_SC_NC = 2
_SC_NS = 16
_SC_NW = _SC_NC * _SC_NS
_SC_L = 16


@functools.cache
def _make_sc_countsum(n):
    per_w = n // _SC_NW
    steps = per_w // _SC_L
    mesh = plsc.VectorSubcoreMesh(core_axis_name="c", subcore_axis_name="s")

    @functools.partial(
        pl.kernel,
        mesh=mesh,
        out_type=jax.ShapeDtypeStruct((2, _SC_NW, _SC_L), jnp.float32),
        scratch_types=[
            pltpu.VMEM((per_w,), jnp.float32),
            pltpu.VMEM((_SC_L,), jnp.float32),
        ],
    )
    def countsum(loss_hbm, t_hbm, out_hbm, chunk, vec):
        cid = jax.lax.axis_index("c")
        sid = jax.lax.axis_index("s")
        wid = sid * _SC_NC + cid
        pltpu.sync_copy(loss_hbm.at[pl.ds(wid * per_w, per_w)], chunk)
        pltpu.sync_copy(t_hbm, vec)
        t = vec[...]
        thr = jnp.full((_SC_L,), _THRESH, jnp.float32)
        zero = jnp.zeros((_SC_L,), jnp.float32)
        one = jnp.full((_SC_L,), 1.0, jnp.float32)

        def body(i, carry):
            c_acc, s_acc = carry
            x = chunk[pl.ds(i * _SC_L, _SC_L)]
            keep = (x <= thr) & (x > t)
            return (
                c_acc + jnp.where(keep, one, zero),
                s_acc + jnp.where(keep, x, zero),
            )

        c_acc, s_acc = jax.lax.fori_loop(0, steps, body, (zero, zero))
        vec[...] = c_acc
        pltpu.sync_copy(vec, out_hbm.at[0, wid])
        vec[...] = s_acc
        pltpu.sync_copy(vec, out_hbm.at[1, wid])

    return countsum


def _select_body(loss_ref, kp_ref, out_ref):
    x = loss_ref[...]
    kp = kp_ref[0]
    in_s = x <= _THRESH

    def it(_, carry):
        lo, hi = carry
        mid = 0.5 * (lo + hi)
        f = jnp.sum((in_s & (x > mid)).astype(jnp.float32))
        gt = f > kp
        return jnp.where(gt, mid, lo), jnp.where(gt, hi, mid)

    lo, hi = jax.lax.fori_loop(
        0, 50, it, (jnp.float32(-1.0), jnp.float32(_THRESH))
    )
    sel = in_s & (x > hi)
    fhi = jnp.sum(sel.astype(jnp.float32))
    shi = jnp.sum(jnp.where(sel, x, 0.0))
    out_ref[0] = shi + (kp - fhi) * hi


def _run_ce_stats(logits, labels):
    B, C, H, W = logits.shape
    return pl.pallas_call(
        _ce_stats_body,
        grid=(B, H // _BH),
        in_specs=[
            pl.BlockSpec((1, C, _BH, W), lambda b, h: (b, 0, h, 0)),
            pl.BlockSpec((1, _BH, W), lambda b, h: (b, h, 0)),
        ],
        out_specs=pl.BlockSpec(memory_space=pltpu.SMEM),
        out_shape=jax.ShapeDtypeStruct((2,), jnp.float32),
        compiler_params=pltpu.CompilerParams(
            dimension_semantics=("arbitrary", "arbitrary")
        ),
    )(logits, labels)


def _topk_mean(logits, labels, cnt, ssum, n_min):
    B, C, H, W = logits.shape
    loss = pl.pallas_call(
        _ce_loss_body,
        grid=(B, H // _BH),
        in_specs=[
            pl.BlockSpec((1, C, _BH, W), lambda b, h: (b, 0, h, 0)),
            pl.BlockSpec((1, _BH, W), lambda b, h: (b, h, 0)),
        ],
        out_specs=pl.BlockSpec((1, _BH, W), lambda b, h: (b, h, 0)),
        out_shape=jax.ShapeDtypeStruct((B, H, W), jnp.float32),
        compiler_params=pltpu.CompilerParams(
            dimension_semantics=("arbitrary", "arbitrary")
        ),
    )(logits, labels)
    loss2d = loss.reshape(B * H, W)
    kp = (jnp.float32(n_min) - cnt).reshape(1)
    rest = pl.pallas_call(
        _select_body,
        in_specs=[
            pl.BlockSpec(loss2d.shape, lambda: (0, 0)),
            pl.BlockSpec(memory_space=pltpu.SMEM),
        ],
        out_specs=pl.BlockSpec(memory_space=pltpu.SMEM),
        out_shape=jax.ShapeDtypeStruct((1,), jnp.float32),
    )(loss2d, kp)
    return (ssum + rest[0]) / jnp.float32(n_min)


def kernel(logits, labels):
    B, C, H, W = logits.shape
    labels = labels.astype(jnp.int32)
    n = B * H * W
    n_min = int(_NMIN_FRAC * n)
    stats = _run_ce_stats(logits, labels)
    cnt, ssum = stats[0], stats[1]
    mean_thresh = ssum / jnp.maximum(cnt, 1.0)
    return jax.lax.cond(
        cnt > jnp.float32(n_min),
        lambda: mean_thresh,
        lambda: _topk_mean(logits, labels, cnt, ssum, n_min),
    )


# speedup vs baseline: 1.1548x; 1.0004x over previous
import numpy as np
import jax
import jax.numpy as jnp
from jax.experimental import pallas as pl
from jax.experimental.pallas import tpu as pltpu

_THRESH = float(-np.log(0.7))
_NMIN_FRAC = 0.1
_IGNORE = 255

_BH = 64


def _ce_loss_tile(z_ref, lab_ref):
    C = z_ref.shape[1]
    lab = lab_ref[0]
    m = z_ref[0, 0]
    for c in range(1, C):
        m = jnp.maximum(m, z_ref[0, c])
    s = jnp.zeros_like(m)
    picked = jnp.zeros_like(m)
    for c in range(C):
        zc = z_ref[0, c]
        s = s + jnp.exp(zc - m)
        picked = jnp.where(lab == c, zc, picked)
    loss = m + jnp.log(s) - picked
    return jnp.where(lab == _IGNORE, 0.0, loss)


def _ce_stats_body(z_ref, lab_ref, out_ref):
    loss = _ce_loss_tile(z_ref, lab_ref)
    mask = loss > _THRESH
    c = jnp.sum(mask.astype(jnp.float32))
    sm = jnp.sum(jnp.where(mask, loss, 0.0))
    first = (pl.program_id(0) == 0) & (pl.program_id(1) == 0)

    @pl.when(first)
    def _():
        out_ref[0] = 0.0
        out_ref[1] = 0.0

    out_ref[0] += c
    out_ref[1] += sm


def _ce_loss_body(z_ref, lab_ref, out_ref):
    out_ref[0] = _ce_loss_tile(z_ref, lab_ref)


def _select_body(loss_ref, kp_ref, out_ref):
    x = loss_ref[...]
    kp = kp_ref[0]
    in_s = x <= _THRESH

    def it(_, carry):
        lo, hi = carry
        mid = 0.5 * (lo + hi)
        f = jnp.sum((in_s & (x > mid)).astype(jnp.float32))
        gt = f > kp
        return jnp.where(gt, mid, lo), jnp.where(gt, hi, mid)

    lo, hi = jax.lax.fori_loop(
        0, 50, it, (jnp.float32(-1.0), jnp.float32(_THRESH))
    )
    sel = in_s & (x > hi)
    fhi = jnp.sum(sel.astype(jnp.float32))
    shi = jnp.sum(jnp.where(sel, x, 0.0))
    out_ref[0] = shi + (kp - fhi) * hi


def _run_ce_stats(logits, labels):
    B, C, H, W = logits.shape
    return pl.pallas_call(
        _ce_stats_body,
        grid=(B, H // _BH),
        in_specs=[
            pl.BlockSpec((1, C, _BH, W), lambda b, h: (b, 0, h, 0)),
            pl.BlockSpec((1, _BH, W), lambda b, h: (b, h, 0)),
        ],
        out_specs=pl.BlockSpec(memory_space=pltpu.SMEM),
        out_shape=jax.ShapeDtypeStruct((2,), jnp.float32),
        compiler_params=pltpu.CompilerParams(
            dimension_semantics=("arbitrary", "arbitrary")
        ),
    )(logits, labels)


def _topk_mean(logits, labels, cnt, ssum, n_min):
    B, C, H, W = logits.shape
    loss = pl.pallas_call(
        _ce_loss_body,
        grid=(B, H // _BH),
        in_specs=[
            pl.BlockSpec((1, C, _BH, W), lambda b, h: (b, 0, h, 0)),
            pl.BlockSpec((1, _BH, W), lambda b, h: (b, h, 0)),
        ],
        out_specs=pl.BlockSpec((1, _BH, W), lambda b, h: (b, h, 0)),
        out_shape=jax.ShapeDtypeStruct((B, H, W), jnp.float32),
        compiler_params=pltpu.CompilerParams(
            dimension_semantics=("arbitrary", "arbitrary")
        ),
    )(logits, labels)
    loss2d = loss.reshape(B * H, W)
    kp = (jnp.float32(n_min) - cnt).reshape(1)
    rest = pl.pallas_call(
        _select_body,
        in_specs=[
            pl.BlockSpec(loss2d.shape, lambda: (0, 0)),
            pl.BlockSpec(memory_space=pltpu.SMEM),
        ],
        out_specs=pl.BlockSpec(memory_space=pltpu.SMEM),
        out_shape=jax.ShapeDtypeStruct((1,), jnp.float32),
    )(loss2d, kp)
    return (ssum + rest[0]) / jnp.float32(n_min)


def kernel(logits, labels):
    B, C, H, W = logits.shape
    labels = labels.astype(jnp.int32)
    n = B * H * W
    n_min = int(_NMIN_FRAC * n)
    stats = _run_ce_stats(logits, labels)
    cnt, ssum = stats[0], stats[1]
    mean_thresh = ssum / jnp.maximum(cnt, 1.0)
    return jax.lax.cond(
        cnt > jnp.float32(n_min),
        lambda: mean_thresh,
        lambda: _topk_mean(logits, labels, cnt, ssum, n_min),
    )
